# speedup vs baseline: 1.0683x; 1.0683x over previous
import jax
import jax.numpy as jnp
from jax.experimental import pallas as pl

ALPHA = 8


def _pack_body(x_ref, slow_ref, fast_ref):
    g = pl.program_id(1)
    fast_ref[...] = x_ref[...]
    slow_ref[0, 0] = x_ref[0, g]


def kernel(x):
    C, T, H, W = x.shape
    G = T // ALPHA
    slow, fast = pl.pallas_call(
        _pack_body,
        grid=(C, G),
        in_specs=[
            pl.BlockSpec((1, ALPHA, H, W), lambda c, g: (c, g, 0, 0)),
        ],
        out_specs=[
            pl.BlockSpec((1, 1, H, W), lambda c, g: (c, g, 0, 0)),
            pl.BlockSpec((1, ALPHA, H, W), lambda c, g: (c, g, 0, 0)),
        ],
        out_shape=[
            jax.ShapeDtypeStruct((C, G, H, W), x.dtype),
            jax.ShapeDtypeStruct((C, T, H, W), x.dtype),
        ],
    )(x)
    return (slow, fast)
